# baseline (speedup 1.0000x reference)
import functools

import jax
import jax.numpy as jnp
from jax import lax
from jax.experimental import pallas as pl
from jax.experimental.pallas import tpu as pltpu
from jax.experimental.pallas import tpu_sc as plsc

_N = 4194304
_NC = 2
_NS = 16
_NW = _NC * _NS
_PER_W = _N // _NW
_CHUNK = 32768
_NCHUNK = _PER_W // _CHUNK
_L = 16
_UNROLL = 8

_mesh = plsc.VectorSubcoreMesh(core_axis_name="c", subcore_axis_name="s")


@functools.partial(
    pl.kernel,
    mesh=_mesh,
    out_type=jax.ShapeDtypeStruct((_N,), jnp.float32),
    scratch_types=[
        pltpu.VMEM((2, _CHUNK), jnp.float32),
        pltpu.VMEM((_L,), jnp.float32),
        pltpu.VMEM((_L,), jnp.float32),
        pltpu.SemaphoreType.DMA,
        pltpu.SemaphoreType.DMA,
        pltpu.SemaphoreType.DMA,
        pltpu.SemaphoreType.DMA,
    ],
)
def _affine_sc(x_hbm, scale_hbm, shift_hbm, out_hbm, buf, scv, shv,
               in_sem0, in_sem1, out_sem0, out_sem1):
    wid = lax.axis_index("s") * _NC + lax.axis_index("c")
    base = wid * _PER_W

    pltpu.sync_copy(scale_hbm, scv)
    pltpu.sync_copy(shift_hbm, shv)
    s = scv[...]
    t = shv[...]

    in_sems = (in_sem0, in_sem1)
    out_sems = (out_sem0, out_sem1)
    cp_in = [None, None]
    cp_out = [None, None]

    cp_in[0] = pltpu.async_copy(
        x_hbm.at[pl.ds(base, _CHUNK)], buf.at[0], in_sems[0])

    for i in range(_NCHUNK):
        p = i % 2
        if i + 1 < _NCHUNK:
            q = (i + 1) % 2
            if i >= 1:
                cp_out[q].wait()
            cp_in[q] = pltpu.async_copy(
                x_hbm.at[pl.ds(base + (i + 1) * _CHUNK, _CHUNK)],
                buf.at[q], in_sems[q])
        cp_in[p].wait()

        def body(j, _, p=p):
            b0 = j * (_L * _UNROLL)
            for u in range(_UNROLL):
                sl = pl.ds(b0 + u * _L, _L)
                buf[p, sl] = buf[p, sl] * s + t
            return 0

        lax.fori_loop(0, _CHUNK // (_L * _UNROLL), body, 0)

        cp_out[p] = pltpu.async_copy(
            buf.at[p], out_hbm.at[pl.ds(base + i * _CHUNK, _CHUNK)],
            out_sems[p])

    cp_out[(_NCHUNK - 2) % 2].wait()
    cp_out[(_NCHUNK - 1) % 2].wait()


def kernel(x, head, scale, shift):
    del head
    scv = jnp.full((_L,), scale, dtype=jnp.float32)
    shv = jnp.full((_L,), shift, dtype=jnp.float32)
    return _affine_sc(x, scv, shv)
